# baseline (speedup 1.0000x reference)
import functools

import jax
import jax.numpy as jnp
from jax import lax
from jax.experimental import pallas as pl
from jax.experimental.pallas import tpu as pltpu
from jax.experimental.pallas import tpu_sc as plsc

N_NODES = 10000
N_EDGES = 320000
D_NODE = 128
D_EDGE = 16
HIDDEN = 128
D_OUT = 128

NC = 2
NS = 16
L = 16
NW = NC * NS

EPH = N_EDGES // NC
CH = 16000
NCH = EPH // CH
CB = CH // 128
NB = N_EDGES // 128
VPC = CH // L
N_PAD = 10240
CR = N_PAD // 128


NACC = 2


def _sc_body(ei_hbm, ea_hbm, psums_hbm, pcounts_hbm,
             idx0_v, idx1_v, dat0_v, dat1_v,
             acc0_v, acc1_v, counts_v,
             isem0, isem1, dsem0, dsem1):
    cid = lax.axis_index("c")
    sid = lax.axis_index("s")
    wid = cid * NS + sid

    bbase = cid * (NB // NC)

    idx_bufs = (idx0_v, idx1_v)
    dat_bufs = (dat0_v, dat1_v)
    isems = (isem0, isem1)
    dsems = (dsem0, dsem1)

    rt = sid >> 3
    rr = sid & 7

    def start_chunk(k, slot):
        di = pltpu.async_copy(
            ei_hbm.at[pl.ds(bbase + k * CB, CB)], idx_bufs[slot],
            isems[slot])
        dd = pltpu.async_copy(
            ea_hbm.at[rt, pl.ds(bbase + k * CB, CB), rr],
            dat_bufs[slot], dsems[slot])
        return di, dd

    descs = [None, None]
    descs[0] = start_chunk(0, 0)

    zero_row = jnp.zeros((L,), jnp.float32)
    accs = (acc0_v, acc1_v)

    def z_acc(i, c):
        for a in accs:
            a[i >> 3, pl.ds((i & 7) * L, L)] = zero_row
        counts_v[i >> 3, pl.ds((i & 7) * L, L)] = zero_row
        return c
    lax.fori_loop(0, CR * 8, z_acc, 0, unroll=8)

    ho = sid * 62 + jnp.minimum(sid, 8)
    hn = jnp.where(sid < 8, 63, 62)
    ones = jnp.ones((L,), jnp.float32)

    for k in range(NCH):
        if k + 1 < NCH:
            descs[(k + 1) % 2] = start_chunk(k + 1, (k + 1) % 2)
        di, dd = descs[k % 2]
        di.wait()
        dd.wait()
        idx_b = idx_bufs[k % 2]
        dat_b = dat_bufs[k % 2]

        def scat(j, c):
            pairs = [(idx_b[j, 1, pl.ds(i * L, L)], dat_b[j, pl.ds(i * L, L)])
                     for i in range(8)]
            for i, (idx16, val16) in enumerate(pairs):
                plsc.addupdate_scatter(
                    accs[i % NACC], [idx16 >> 7, idx16 & 127], val16)
            return c
        lax.fori_loop(0, CB, scat, 0, unroll=2)

        def hist(j, c):
            m = ho + j
            idx16 = idx_b[m >> 3, 1, pl.ds((m & 7) * L, L)]
            mask = jnp.broadcast_to(j < hn, (L,))
            plsc.addupdate_scatter(
                counts_v, [idx16 >> 7, idx16 & 127], ones, mask=mask)
            return c
        lax.fori_loop(0, 63, hist, 0, unroll=8)

    for q in range(NACC):
        pltpu.sync_copy(accs[q], psums_hbm.at[q * NW + wid])
    pltpu.sync_copy(counts_v, pcounts_hbm.at[wid])


_sc_scatter = functools.partial(
    pl.kernel,
    out_type=[
        jax.ShapeDtypeStruct((NACC * NW, CR, 128), jnp.float32),
        jax.ShapeDtypeStruct((NW, CR, 128), jnp.float32),
    ],
    mesh=plsc.VectorSubcoreMesh(core_axis_name="c", subcore_axis_name="s",
                                num_cores=NC, num_subcores=NS),
    compiler_params=pltpu.CompilerParams(needs_layout_passes=False,
                                         use_tc_tiling_on_sc=False),
    scratch_types=[
        pltpu.VMEM((CB, 2, 128), jnp.int32),
        pltpu.VMEM((CB, 2, 128), jnp.int32),
        pltpu.VMEM((CB, 128), jnp.float32),
        pltpu.VMEM((CB, 128), jnp.float32),
        pltpu.VMEM((CR, 128), jnp.float32),
        pltpu.VMEM((CR, 128), jnp.float32),
        pltpu.VMEM((CR, 128), jnp.float32),
        pltpu.SemaphoreType.DMA,
        pltpu.SemaphoreType.DMA,
        pltpu.SemaphoreType.DMA,
        pltpu.SemaphoreType.DMA,
    ],
)(_sc_body)


_B = 1024


def _mlp_a_body(x_ref, w1_ref, b1_ref, hx_ref):
    hx_ref[...] = jnp.dot(x_ref[...], w1_ref[:D_NODE, :],
                          preferred_element_type=jnp.float32) + b1_ref[...]


_mlp_a = pl.pallas_call(
    _mlp_a_body,
    out_shape=jax.ShapeDtypeStruct((N_NODES, HIDDEN), jnp.float32),
    grid=(pl.cdiv(N_NODES, _B),),
    in_specs=[
        pl.BlockSpec((_B, D_NODE), lambda i: (i, 0)),
        pl.BlockSpec((D_NODE + D_EDGE, HIDDEN), lambda i: (0, 0)),
        pl.BlockSpec((1, HIDDEN), lambda i: (0, 0)),
    ],
    out_specs=pl.BlockSpec((_B, HIDDEN), lambda i: (i, 0)),
)


def _mlp_b_body(hx_ref, ps_ref, pc_ref, w1_ref, w2_ref, b2_ref, o_ref):
    ps = ps_ref[...].reshape(NACC * NW, _B)
    sums_t = ps[:NS]
    for t in range(1, NACC * NC):
        sums_t = sums_t + ps[t * NS:(t + 1) * NS]
    counts = jnp.sum(pc_ref[...].reshape(NW, _B), axis=0)
    inv = 1.0 / jnp.maximum(counts, 1.0)
    mean_t = sums_t * inv[None, :]
    hm = lax.dot_general(mean_t, w1_ref[D_NODE:, :],
                         dimension_numbers=(((0,), (0,)), ((), ())),
                         preferred_element_type=jnp.float32)
    h = jnp.maximum(hx_ref[...] + hm, 0.0)
    o_ref[...] = jnp.dot(h, w2_ref[...],
                         preferred_element_type=jnp.float32) + b2_ref[...]


_mlp_b = pl.pallas_call(
    _mlp_b_body,
    out_shape=jax.ShapeDtypeStruct((N_NODES, D_OUT), jnp.float32),
    grid=(pl.cdiv(N_NODES, _B),),
    in_specs=[
        pl.BlockSpec((_B, HIDDEN), lambda i: (i, 0)),
        pl.BlockSpec((NACC * NW, _B // 128, 128), lambda i: (0, i, 0)),
        pl.BlockSpec((NW, _B // 128, 128), lambda i: (0, i, 0)),
        pl.BlockSpec((D_NODE + D_EDGE, HIDDEN), lambda i: (0, 0)),
        pl.BlockSpec((HIDDEN, D_OUT), lambda i: (0, 0)),
        pl.BlockSpec((1, D_OUT), lambda i: (0, 0)),
    ],
    out_specs=pl.BlockSpec((_B, D_OUT), lambda i: (i, 0)),
)


@jax.jit
def kernel(x, edge_index, edge_attr, W1, b1, W2, b2):
    ei_blocks = edge_index.reshape(2, NB, 128).transpose(1, 0, 2)
    ea_blocks = edge_attr.T.reshape(2, 8, NB, 128).transpose(0, 2, 1, 3)
    psums, pcounts = _sc_scatter(ei_blocks, ea_blocks)
    hx = _mlp_a(x, W1, b1.reshape(1, HIDDEN))
    return _mlp_b(hx, psums, pcounts, W1, W2, b2.reshape(1, D_OUT))


# speedup vs baseline: 1.0554x; 1.0554x over previous
import functools

import jax
import jax.numpy as jnp
from jax import lax
from jax.experimental import pallas as pl
from jax.experimental.pallas import tpu as pltpu
from jax.experimental.pallas import tpu_sc as plsc

N_NODES = 10000
N_EDGES = 320000
D_NODE = 128
D_EDGE = 16
HIDDEN = 128
D_OUT = 128

NC = 2
NS = 16
L = 16
NW = NC * NS

EPH = N_EDGES // NC
CH = 16000
NCH = EPH // CH
CB = CH // 128
NB = N_EDGES // 128
VPC = CH // L
N_PAD = 10240
CR = N_PAD // 128


NACC = 2


def _sc_body(ei_hbm, ea_hbm, psums_hbm, pcounts_hbm,
             idx0_v, idx1_v, dat0_v, dat1_v,
             acc0_v, acc1_v, counts_v,
             isem0, isem1, dsem0, dsem1):
    cid = lax.axis_index("c")
    sid = lax.axis_index("s")
    wid = cid * NS + sid

    bbase = cid * (NB // NC)

    idx_bufs = (idx0_v, idx1_v)
    dat_bufs = (dat0_v, dat1_v)
    isems = (isem0, isem1)
    dsems = (dsem0, dsem1)

    rt = sid >> 3
    rr = sid & 7

    def start_chunk(k, slot):
        di = pltpu.async_copy(
            ei_hbm.at[pl.ds(bbase + k * CB, CB), 1], idx_bufs[slot],
            isems[slot])
        dd = pltpu.async_copy(
            ea_hbm.at[rt, pl.ds(bbase + k * CB, CB), rr],
            dat_bufs[slot], dsems[slot])
        return di, dd

    descs = [None, None]
    descs[0] = start_chunk(0, 0)

    zero_row = jnp.zeros((L,), jnp.float32)
    accs = (acc0_v, acc1_v)

    def z_acc(i, c):
        for a in accs:
            a[i >> 3, pl.ds((i & 7) * L, L)] = zero_row
        counts_v[i >> 3, pl.ds((i & 7) * L, L)] = zero_row
        return c
    lax.fori_loop(0, CR * 8, z_acc, 0, unroll=8)

    ho = sid * 62 + jnp.minimum(sid, 8)
    hn = jnp.where(sid < 8, 63, 62)
    ones = jnp.ones((L,), jnp.float32)

    for k in range(NCH):
        if k + 1 < NCH:
            descs[(k + 1) % 2] = start_chunk(k + 1, (k + 1) % 2)
        di, dd = descs[k % 2]
        di.wait()
        dd.wait()
        idx_b = idx_bufs[k % 2]
        dat_b = dat_bufs[k % 2]

        def scat(j, c):
            pairs = [(idx_b[j, pl.ds(i * L, L)], dat_b[j, pl.ds(i * L, L)])
                     for i in range(8)]
            for i, (idx16, val16) in enumerate(pairs):
                plsc.addupdate_scatter(
                    accs[i % NACC], [idx16 >> 7, idx16 & 127], val16)
            return c
        lax.fori_loop(0, CB, scat, 0, unroll=2)

        def hist(j, c):
            m = ho + j
            idx16 = idx_b[m >> 3, pl.ds((m & 7) * L, L)]
            mask = jnp.broadcast_to(j < hn, (L,))
            plsc.addupdate_scatter(
                counts_v, [idx16 >> 7, idx16 & 127], ones, mask=mask)
            return c
        lax.fori_loop(0, 63, hist, 0, unroll=8)

    for q in range(NACC):
        pltpu.sync_copy(accs[q], psums_hbm.at[q * NW + wid])
    pltpu.sync_copy(counts_v, pcounts_hbm.at[wid])


_sc_scatter = functools.partial(
    pl.kernel,
    out_type=[
        jax.ShapeDtypeStruct((NACC * NW, CR, 128), jnp.float32),
        jax.ShapeDtypeStruct((NW, CR, 128), jnp.float32),
    ],
    mesh=plsc.VectorSubcoreMesh(core_axis_name="c", subcore_axis_name="s",
                                num_cores=NC, num_subcores=NS),
    compiler_params=pltpu.CompilerParams(needs_layout_passes=False,
                                         use_tc_tiling_on_sc=False),
    scratch_types=[
        pltpu.VMEM((CB, 128), jnp.int32),
        pltpu.VMEM((CB, 128), jnp.int32),
        pltpu.VMEM((CB, 128), jnp.float32),
        pltpu.VMEM((CB, 128), jnp.float32),
        pltpu.VMEM((CR, 128), jnp.float32),
        pltpu.VMEM((CR, 128), jnp.float32),
        pltpu.VMEM((CR, 128), jnp.float32),
        pltpu.SemaphoreType.DMA,
        pltpu.SemaphoreType.DMA,
        pltpu.SemaphoreType.DMA,
        pltpu.SemaphoreType.DMA,
    ],
)(_sc_body)


_B = 1024


def _mlp_a_body(x_ref, w1_ref, b1_ref, hx_ref):
    hx_ref[...] = jnp.dot(x_ref[...], w1_ref[:D_NODE, :],
                          preferred_element_type=jnp.float32) + b1_ref[...]


_mlp_a = pl.pallas_call(
    _mlp_a_body,
    out_shape=jax.ShapeDtypeStruct((N_NODES, HIDDEN), jnp.float32),
    grid=(pl.cdiv(N_NODES, _B),),
    in_specs=[
        pl.BlockSpec((_B, D_NODE), lambda i: (i, 0)),
        pl.BlockSpec((D_NODE + D_EDGE, HIDDEN), lambda i: (0, 0)),
        pl.BlockSpec((1, HIDDEN), lambda i: (0, 0)),
    ],
    out_specs=pl.BlockSpec((_B, HIDDEN), lambda i: (i, 0)),
)


def _mlp_b_body(hx_ref, ps_ref, pc_ref, w1_ref, w2_ref, b2_ref, o_ref):
    ps = ps_ref[...].reshape(NACC * NW, _B)
    sums_t = ps[:NS]
    for t in range(1, NACC * NC):
        sums_t = sums_t + ps[t * NS:(t + 1) * NS]
    counts = jnp.sum(pc_ref[...].reshape(NW, _B), axis=0)
    inv = 1.0 / jnp.maximum(counts, 1.0)
    mean_t = sums_t * inv[None, :]
    hm = lax.dot_general(mean_t, w1_ref[D_NODE:, :],
                         dimension_numbers=(((0,), (0,)), ((), ())),
                         preferred_element_type=jnp.float32)
    h = jnp.maximum(hx_ref[...] + hm, 0.0)
    o_ref[...] = jnp.dot(h, w2_ref[...],
                         preferred_element_type=jnp.float32) + b2_ref[...]


_mlp_b = pl.pallas_call(
    _mlp_b_body,
    out_shape=jax.ShapeDtypeStruct((N_NODES, D_OUT), jnp.float32),
    grid=(pl.cdiv(N_NODES, _B),),
    in_specs=[
        pl.BlockSpec((_B, HIDDEN), lambda i: (i, 0)),
        pl.BlockSpec((NACC * NW, _B // 128, 128), lambda i: (0, i, 0)),
        pl.BlockSpec((NW, _B // 128, 128), lambda i: (0, i, 0)),
        pl.BlockSpec((D_NODE + D_EDGE, HIDDEN), lambda i: (0, 0)),
        pl.BlockSpec((HIDDEN, D_OUT), lambda i: (0, 0)),
        pl.BlockSpec((1, D_OUT), lambda i: (0, 0)),
    ],
    out_specs=pl.BlockSpec((_B, D_OUT), lambda i: (i, 0)),
)


@jax.jit
def kernel(x, edge_index, edge_attr, W1, b1, W2, b2):
    ei_blocks = edge_index.reshape(2, NB, 128).transpose(1, 0, 2)
    ea_blocks = edge_attr.T.reshape(2, 8, NB, 128).transpose(0, 2, 1, 3)
    psums, pcounts = _sc_scatter(ei_blocks, ea_blocks)
    hx = _mlp_a(x, W1, b1.reshape(1, HIDDEN))
    return _mlp_b(hx, psums, pcounts, W1, W2, b2.reshape(1, D_OUT))


# speedup vs baseline: 1.0996x; 1.0419x over previous
import functools

import jax
import jax.numpy as jnp
from jax import lax
from jax.experimental import pallas as pl
from jax.experimental.pallas import tpu as pltpu
from jax.experimental.pallas import tpu_sc as plsc

N_NODES = 10000
N_EDGES = 320000
D_NODE = 128
D_EDGE = 16
HIDDEN = 128
D_OUT = 128

NC = 2
NS = 16
L = 16
NW = NC * NS

EPH = N_EDGES // NC
CH = 16000
NCH = EPH // CH
CB = CH // 128
NB = N_EDGES // 128
VPC = CH // L
N_PAD = 10240
CR = N_PAD // 128


NACC = 2


def _sc_body(ei_hbm, ea_hbm, psums_hbm, pcounts_hbm,
             idx0_v, idx1_v, dat0_v, dat1_v,
             acc0_v, acc1_v, counts_v,
             isem0, isem1, dsem0, dsem1):
    cid = lax.axis_index("c")
    sid = lax.axis_index("s")
    wid = cid * NS + sid

    bbase = cid * (NB // NC)

    idx_bufs = (idx0_v, idx1_v)
    dat_bufs = (dat0_v, dat1_v)
    isems = (isem0, isem1)
    dsems = (dsem0, dsem1)

    rt = sid >> 3
    rr = sid & 7

    def start_chunk(k, slot):
        di = pltpu.async_copy(
            ei_hbm.at[pl.ds(bbase + k * CB, CB), 1], idx_bufs[slot],
            isems[slot])
        dd = pltpu.async_copy(
            ea_hbm.at[rt, pl.ds(bbase + k * CB, CB), rr],
            dat_bufs[slot], dsems[slot])
        return di, dd

    descs = [None, None]
    descs[0] = start_chunk(0, 0)

    zero_row = jnp.zeros((L,), jnp.float32)
    accs = (acc0_v, acc1_v)

    def z_acc(i, c):
        for a in accs:
            a[i >> 3, pl.ds((i & 7) * L, L)] = zero_row
        counts_v[i >> 3, pl.ds((i & 7) * L, L)] = zero_row
        return c
    lax.fori_loop(0, CR * 8, z_acc, 0, unroll=8)

    ho = sid * 62 + jnp.minimum(sid, 8)
    hn = jnp.where(sid < 8, 63, 62)
    ones = jnp.ones((L,), jnp.float32)

    for k in range(NCH):
        if k + 1 < NCH:
            descs[(k + 1) % 2] = start_chunk(k + 1, (k + 1) % 2)
        di, dd = descs[k % 2]
        di.wait()
        dd.wait()
        idx_b = idx_bufs[k % 2]
        dat_b = dat_bufs[k % 2]

        def scat(j, c):
            pairs = [(idx_b[j, pl.ds(i * L, L)], dat_b[j, pl.ds(i * L, L)])
                     for i in range(8)]
            for i, (idx16, val16) in enumerate(pairs):
                plsc.addupdate_scatter(
                    accs[i % NACC], [idx16 >> 7, idx16 & 127], val16)
            return c
        lax.fori_loop(0, CB, scat, 0, unroll=2)

        def hist(j, c):
            m = ho + j
            idx16 = idx_b[m >> 3, pl.ds((m & 7) * L, L)]
            mask = jnp.broadcast_to(j < hn, (L,))
            plsc.addupdate_scatter(
                counts_v, [idx16 >> 7, idx16 & 127], ones, mask=mask)
            return c
        lax.fori_loop(0, 63, hist, 0, unroll=8)

    for q in range(NACC):
        pltpu.sync_copy(accs[q], psums_hbm.at[q * NW + wid])
    pltpu.sync_copy(counts_v, pcounts_hbm.at[wid])


_sc_scatter = functools.partial(
    pl.kernel,
    out_type=[
        jax.ShapeDtypeStruct((NACC * NW, CR, 128), jnp.float32),
        jax.ShapeDtypeStruct((NW, CR, 128), jnp.float32),
    ],
    mesh=plsc.VectorSubcoreMesh(core_axis_name="c", subcore_axis_name="s",
                                num_cores=NC, num_subcores=NS),
    compiler_params=pltpu.CompilerParams(needs_layout_passes=False,
                                         use_tc_tiling_on_sc=False),
    scratch_types=[
        pltpu.VMEM((CB, 128), jnp.int32),
        pltpu.VMEM((CB, 128), jnp.int32),
        pltpu.VMEM((CB, 128), jnp.float32),
        pltpu.VMEM((CB, 128), jnp.float32),
        pltpu.VMEM((CR, 128), jnp.float32),
        pltpu.VMEM((CR, 128), jnp.float32),
        pltpu.VMEM((CR, 128), jnp.float32),
        pltpu.SemaphoreType.DMA,
        pltpu.SemaphoreType.DMA,
        pltpu.SemaphoreType.DMA,
        pltpu.SemaphoreType.DMA,
    ],
)(_sc_body)


_B = 2048


def _mlp_a_body(x_ref, w1_ref, b1_ref, hx_ref):
    hx_ref[...] = jnp.dot(x_ref[...], w1_ref[:D_NODE, :],
                          preferred_element_type=jnp.float32) + b1_ref[...]


_mlp_a = pl.pallas_call(
    _mlp_a_body,
    out_shape=jax.ShapeDtypeStruct((N_NODES, HIDDEN), jnp.float32),
    grid=(pl.cdiv(N_NODES, _B),),
    in_specs=[
        pl.BlockSpec((_B, D_NODE), lambda i: (i, 0)),
        pl.BlockSpec((D_NODE + D_EDGE, HIDDEN), lambda i: (0, 0)),
        pl.BlockSpec((1, HIDDEN), lambda i: (0, 0)),
    ],
    out_specs=pl.BlockSpec((_B, HIDDEN), lambda i: (i, 0)),
)


def _mlp_b_body(hx_ref, ps_ref, pc_ref, w1_ref, w2_ref, b2_ref, o_ref):
    ps = ps_ref[...].reshape(NACC * NW, _B)
    sums_t = ps[:NS]
    for t in range(1, NACC * NC):
        sums_t = sums_t + ps[t * NS:(t + 1) * NS]
    counts = jnp.sum(pc_ref[...].reshape(NW, _B), axis=0)
    inv = 1.0 / jnp.maximum(counts, 1.0)
    mean_t = sums_t * inv[None, :]
    hm = lax.dot_general(mean_t, w1_ref[D_NODE:, :],
                         dimension_numbers=(((0,), (0,)), ((), ())),
                         preferred_element_type=jnp.float32)
    h = jnp.maximum(hx_ref[...] + hm, 0.0)
    o_ref[...] = jnp.dot(h, w2_ref[...],
                         preferred_element_type=jnp.float32) + b2_ref[...]


_mlp_b = pl.pallas_call(
    _mlp_b_body,
    out_shape=jax.ShapeDtypeStruct((N_NODES, D_OUT), jnp.float32),
    grid=(pl.cdiv(N_NODES, _B),),
    in_specs=[
        pl.BlockSpec((_B, HIDDEN), lambda i: (i, 0)),
        pl.BlockSpec((NACC * NW, _B // 128, 128), lambda i: (0, i, 0)),
        pl.BlockSpec((NW, _B // 128, 128), lambda i: (0, i, 0)),
        pl.BlockSpec((D_NODE + D_EDGE, HIDDEN), lambda i: (0, 0)),
        pl.BlockSpec((HIDDEN, D_OUT), lambda i: (0, 0)),
        pl.BlockSpec((1, D_OUT), lambda i: (0, 0)),
    ],
    out_specs=pl.BlockSpec((_B, D_OUT), lambda i: (i, 0)),
)


@jax.jit
def kernel(x, edge_index, edge_attr, W1, b1, W2, b2):
    ei_blocks = edge_index.reshape(2, NB, 128).transpose(1, 0, 2)
    ea_blocks = edge_attr.T.reshape(2, 8, NB, 128).transpose(0, 2, 1, 3)
    psums, pcounts = _sc_scatter(ei_blocks, ea_blocks)
    hx = _mlp_a(x, W1, b1.reshape(1, HIDDEN))
    return _mlp_b(hx, psums, pcounts, W1, W2, b2.reshape(1, D_OUT))


# speedup vs baseline: 1.1537x; 1.0492x over previous
import functools

import jax
import jax.numpy as jnp
from jax import lax
from jax.experimental import pallas as pl
from jax.experimental.pallas import tpu as pltpu
from jax.experimental.pallas import tpu_sc as plsc

N_NODES = 10000
N_EDGES = 320000
D_NODE = 128
D_EDGE = 16
HIDDEN = 128
D_OUT = 128

NC = 2
NS = 16
L = 16
NW = NC * NS

EPH = N_EDGES // NC
CH = 16000
NCH = EPH // CH
CB = CH // 128
NB = N_EDGES // 128
VPC = CH // L
N_PAD = 10240
CR = N_PAD // 128


NACC = 2


def _sc_body(ei_hbm, ea_hbm, psums_hbm, pcounts_hbm,
             idx0_v, idx1_v, dat0_v, dat1_v,
             acc0_v, acc1_v, counts_v,
             isem0, isem1, dsem0, dsem1):
    cid = lax.axis_index("c")
    sid = lax.axis_index("s")
    wid = cid * NS + sid

    bbase = cid * (NB // NC)

    idx_bufs = (idx0_v, idx1_v)
    dat_bufs = (dat0_v, dat1_v)
    isems = (isem0, isem1)
    dsems = (dsem0, dsem1)

    rt = sid >> 3
    rr = sid & 7

    def start_chunk(k, slot):
        di = pltpu.async_copy(
            ei_hbm.at[pl.ds(bbase + k * CB, CB), 1], idx_bufs[slot],
            isems[slot])
        dd = pltpu.async_copy(
            ea_hbm.at[rt, pl.ds(bbase + k * CB, CB), rr],
            dat_bufs[slot], dsems[slot])
        return di, dd

    descs = [None, None]
    descs[0] = start_chunk(0, 0)

    zero_row = jnp.zeros((L,), jnp.float32)
    accs = (acc0_v, acc1_v)

    def z_acc(i, c):
        for a in accs:
            a[i >> 3, pl.ds((i & 7) * L, L)] = zero_row
        counts_v[i >> 3, pl.ds((i & 7) * L, L)] = zero_row
        return c
    lax.fori_loop(0, CR * 8, z_acc, 0, unroll=8)

    ho = sid * 62 + jnp.minimum(sid, 8)
    hn = jnp.where(sid < 8, 63, 62)
    ones = jnp.ones((L,), jnp.float32)

    for k in range(NCH):
        if k + 1 < NCH:
            descs[(k + 1) % 2] = start_chunk(k + 1, (k + 1) % 2)
        di, dd = descs[k % 2]
        di.wait()
        dd.wait()
        idx_b = idx_bufs[k % 2]
        dat_b = dat_bufs[k % 2]

        def scat(j, c):
            pairs = [(idx_b[j, pl.ds(i * L, L)], dat_b[j, pl.ds(i * L, L)])
                     for i in range(8)]
            for i, (idx16, val16) in enumerate(pairs):
                plsc.addupdate_scatter(
                    accs[i % NACC], [idx16 >> 7, idx16 & 127], val16)
            return c
        lax.fori_loop(0, CB, scat, 0, unroll=2)

        def hist(j, c):
            m = ho + j
            idx16 = idx_b[m >> 3, pl.ds((m & 7) * L, L)]
            mask = jnp.broadcast_to(j < hn, (L,))
            plsc.addupdate_scatter(
                counts_v, [idx16 >> 7, idx16 & 127], ones, mask=mask)
            return c
        lax.fori_loop(0, 63, hist, 0, unroll=8)

    for q in range(NACC):
        pltpu.sync_copy(accs[q], psums_hbm.at[q * NW + wid])
    pltpu.sync_copy(counts_v, pcounts_hbm.at[wid])


_sc_scatter = functools.partial(
    pl.kernel,
    out_type=[
        jax.ShapeDtypeStruct((NACC * NW, CR, 128), jnp.float32),
        jax.ShapeDtypeStruct((NW, CR, 128), jnp.float32),
    ],
    mesh=plsc.VectorSubcoreMesh(core_axis_name="c", subcore_axis_name="s",
                                num_cores=NC, num_subcores=NS),
    compiler_params=pltpu.CompilerParams(needs_layout_passes=False,
                                         use_tc_tiling_on_sc=False),
    scratch_types=[
        pltpu.VMEM((CB, 128), jnp.int32),
        pltpu.VMEM((CB, 128), jnp.int32),
        pltpu.VMEM((CB, 128), jnp.float32),
        pltpu.VMEM((CB, 128), jnp.float32),
        pltpu.VMEM((CR, 128), jnp.float32),
        pltpu.VMEM((CR, 128), jnp.float32),
        pltpu.VMEM((CR, 128), jnp.float32),
        pltpu.SemaphoreType.DMA,
        pltpu.SemaphoreType.DMA,
        pltpu.SemaphoreType.DMA,
        pltpu.SemaphoreType.DMA,
    ],
)(_sc_body)


_B = 5120


def _mlp_a_body(x_ref, w1_ref, b1_ref, hx_ref):
    hx_ref[...] = jnp.dot(x_ref[...], w1_ref[:D_NODE, :],
                          preferred_element_type=jnp.float32) + b1_ref[...]


_mlp_a = pl.pallas_call(
    _mlp_a_body,
    out_shape=jax.ShapeDtypeStruct((N_NODES, HIDDEN), jnp.float32),
    grid=(pl.cdiv(N_NODES, _B),),
    in_specs=[
        pl.BlockSpec((_B, D_NODE), lambda i: (i, 0)),
        pl.BlockSpec((D_NODE + D_EDGE, HIDDEN), lambda i: (0, 0)),
        pl.BlockSpec((1, HIDDEN), lambda i: (0, 0)),
    ],
    out_specs=pl.BlockSpec((_B, HIDDEN), lambda i: (i, 0)),
)


def _mlp_b_body(hx_ref, ps_ref, pc_ref, w1_ref, w2_ref, b2_ref, o_ref):
    ps = ps_ref[...].reshape(NACC * NW, _B)
    sums_t = ps[:NS]
    for t in range(1, NACC * NC):
        sums_t = sums_t + ps[t * NS:(t + 1) * NS]
    counts = jnp.sum(pc_ref[...].reshape(NW, _B), axis=0)
    inv = 1.0 / jnp.maximum(counts, 1.0)
    mean_t = sums_t * inv[None, :]
    hm = lax.dot_general(mean_t, w1_ref[D_NODE:, :],
                         dimension_numbers=(((0,), (0,)), ((), ())),
                         preferred_element_type=jnp.float32)
    h = jnp.maximum(hx_ref[...] + hm, 0.0)
    o_ref[...] = jnp.dot(h, w2_ref[...],
                         preferred_element_type=jnp.float32) + b2_ref[...]


_mlp_b = pl.pallas_call(
    _mlp_b_body,
    out_shape=jax.ShapeDtypeStruct((N_NODES, D_OUT), jnp.float32),
    grid=(pl.cdiv(N_NODES, _B),),
    in_specs=[
        pl.BlockSpec((_B, HIDDEN), lambda i: (i, 0)),
        pl.BlockSpec((NACC * NW, _B // 128, 128), lambda i: (0, i, 0)),
        pl.BlockSpec((NW, _B // 128, 128), lambda i: (0, i, 0)),
        pl.BlockSpec((D_NODE + D_EDGE, HIDDEN), lambda i: (0, 0)),
        pl.BlockSpec((HIDDEN, D_OUT), lambda i: (0, 0)),
        pl.BlockSpec((1, D_OUT), lambda i: (0, 0)),
    ],
    out_specs=pl.BlockSpec((_B, D_OUT), lambda i: (i, 0)),
)


@jax.jit
def kernel(x, edge_index, edge_attr, W1, b1, W2, b2):
    ei_blocks = edge_index.reshape(2, NB, 128).transpose(1, 0, 2)
    ea_blocks = edge_attr.T.reshape(2, 8, NB, 128).transpose(0, 2, 1, 3)
    psums, pcounts = _sc_scatter(ei_blocks, ea_blocks)
    hx = _mlp_a(x, W1, b1.reshape(1, HIDDEN))
    return _mlp_b(hx, psums, pcounts, W1, W2, b2.reshape(1, D_OUT))
